# baseline (speedup 1.0000x reference)
import functools

import jax
import jax.numpy as jnp
from jax import lax
from jax.experimental import pallas as pl
from jax.experimental.pallas import tpu as pltpu
from jax.experimental.pallas import tpu_sc as plsc

N = 10000
E = 320000
D_IN = 128
D_H = 128
D_Z = 64

NC = 2
NS = 16
NW = NC * NS
L = 16

CH = 128
E_P = 327680
NCHP = E_P // CH
NCH_T = NCHP // NW
NGRP = NCH_T // 8
NPAIR = NCH_T // 2
NPAD = 10240
ROWS_PER_TILE = NPAD // NS

_MESH = plsc.VectorSubcoreMesh(core_axis_name="c", subcore_axis_name="s",
                               num_cores=2, num_subcores=16)


def _wid():
    return lax.axis_index("c") * NS + lax.axis_index("s")


@functools.partial(
    pl.kernel,
    out_type=jax.ShapeDtypeStruct((NC, NPAD, L), jnp.float32),
    mesh=_MESH,
    scratch_types=[
        pltpu.VMEM((NCH_T, CH), jnp.int32),
        pltpu.VMEM((CH, L), jnp.float32),
        pltpu.VMEM((CH, L), jnp.float32),
        pltpu.VMEM_SHARED((NPAD, L), jnp.float32),
        pltpu.SemaphoreType.DMA,
    ],
)
def _deg_sc(dst_hbm, deg_hbm, idx_all, ones_v, zb_v, acc, sem):
    cid = lax.axis_index("c")
    sid = lax.axis_index("s")
    wid = _wid()

    def fill(r, _):
        ones_v[r, :] = jnp.full((L,), 1.0, jnp.float32)
        zb_v[r, :] = jnp.zeros((L,), jnp.float32)
        return 0

    lax.fori_loop(0, CH, fill, 0)
    for k in range(NGRP):
        pltpu.sync_copy(dst_hbm.at[pl.ds((k * NW + wid) * 8, 8)],
                        idx_all.at[pl.ds(k * 8, 8)])
    for k in range(5):
        pltpu.sync_copy(
            zb_v, acc.at[pl.ds(sid * ROWS_PER_TILE + k * CH, CH)])
    plsc.subcore_barrier()

    def group(k, _):
        descs = []
        for j in range(8):
            descs.append(
                pltpu.async_copy(ones_v, acc.at[idx_all.at[k * 8 + j]], sem,
                                 add=True))
        for d in descs:
            d.wait()
        return 0

    lax.fori_loop(0, NGRP, group, 0)
    plsc.subcore_barrier()
    pltpu.sync_copy(
        acc.at[pl.ds(sid * ROWS_PER_TILE, ROWS_PER_TILE)],
        deg_hbm.at[cid, pl.ds(sid * ROWS_PER_TILE, ROWS_PER_TILE)],
    )


@functools.partial(
    pl.kernel,
    out_type=jax.ShapeDtypeStruct((NC, NPAD, D_H), jnp.float32),
    mesh=_MESH,
    scratch_types=[
        pltpu.VMEM((CH,), jnp.int32),
        pltpu.VMEM((CH,), jnp.int32),
        pltpu.VMEM((CH, D_H), jnp.float32),
        pltpu.VMEM((CH, D_H), jnp.float32),
        pltpu.VMEM_SHARED((NPAD, D_H), jnp.float32),
        pltpu.SemaphoreType.DMA,
    ],
)
def _agg_sc(g_hbm, src_hbm, dst_hbm, out_hbm, idx_s, idx_d, rows_v, zb_v,
            acc, sem):
    cid = lax.axis_index("c")
    sid = lax.axis_index("s")
    wid = _wid()

    def fill(r, _):
        for c8 in range(D_H // L):
            zb_v[r, pl.ds(c8 * L, L)] = jnp.zeros((L,), jnp.float32)
        return 0

    lax.fori_loop(0, CH, fill, 0)
    for k in range(5):
        pltpu.sync_copy(
            zb_v, acc.at[pl.ds(sid * ROWS_PER_TILE + k * CH, CH)])
    plsc.subcore_barrier()

    def body(c, _):
        base = (c * NW + wid) * CH
        pltpu.sync_copy(src_hbm.at[pl.ds(base, CH)], idx_s)
        pltpu.sync_copy(dst_hbm.at[pl.ds(base, CH)], idx_d)
        pltpu.async_copy(g_hbm.at[idx_s], rows_v, sem).wait()
        pltpu.sync_copy(rows_v, acc.at[idx_d], add=True)
        return 0

    lax.fori_loop(0, NCH_T, body, 0)
    plsc.subcore_barrier()
    pltpu.sync_copy(
        acc.at[pl.ds(sid * ROWS_PER_TILE, ROWS_PER_TILE)],
        out_hbm.at[cid, pl.ds(sid * ROWS_PER_TILE, ROWS_PER_TILE)],
    )


@functools.partial(
    pl.kernel,
    out_type=jax.ShapeDtypeStruct((E_P * L,), jnp.float32),
    mesh=_MESH,
    scratch_types=[
        pltpu.VMEM((CH,), jnp.int32),
        pltpu.VMEM((CH,), jnp.int32),
        pltpu.VMEM((CH, D_H), jnp.float32),
        pltpu.VMEM((CH, D_H), jnp.float32),
        pltpu.VMEM((CH * L,), jnp.float32),
        pltpu.SemaphoreType.DMA,
    ],
)
def _dec_sc(z_hbm, src_hbm, dst_hbm, q_hbm, idx_s, idx_d, zs_v, zd_v, q_v,
            sem):
    wid = _wid()

    def body(c, _):
        base = (c * NW + wid) * CH
        pltpu.sync_copy(src_hbm.at[pl.ds(base, CH)], idx_s)
        pltpu.sync_copy(dst_hbm.at[pl.ds(base, CH)], idx_d)
        pltpu.async_copy(z_hbm.at[idx_s], zs_v, sem).wait()
        pltpu.async_copy(z_hbm.at[idx_d], zd_v, sem).wait()

        def dot_edge(i, _):
            for u in range(2):
                e = 2 * i + u
                q = zs_v[e, pl.ds(0, L)] * zd_v[e, pl.ds(0, L)]
                for t in range(1, D_Z // L):
                    q = q + zs_v[e, pl.ds(t * L, L)] * zd_v[e, pl.ds(t * L, L)]
                q_v[pl.ds(e * L, L)] = q
            return 0

        lax.fori_loop(0, CH // 2, dot_edge, 0)
        pltpu.sync_copy(q_v, q_hbm.at[pl.ds(base * L, CH * L)])
        return 0

    lax.fori_loop(0, NCH_T, body, 0)


def _t1_body(x_ref, w_ref, d0_ref, d1_ref, g_ref):
    deg = d0_ref[:, 0:1] + d1_ref[:, 0:1] + 1.0
    dinv = lax.rsqrt(jnp.maximum(deg, 1e-12))
    h = jnp.dot(x_ref[...], w_ref[...], preferred_element_type=jnp.float32)
    g_ref[...] = h * dinv


def _t2_body(s0_ref, s1_ref, g1_ref, d0_ref, d1_ref, b1_ref, w_ref, g2_ref):
    deg = d0_ref[:, 0:1] + d1_ref[:, 0:1] + 1.0
    dinv = lax.rsqrt(jnp.maximum(deg, 1e-12))
    h = jnp.maximum(
        dinv * (s0_ref[...] + s1_ref[...] + g1_ref[...]) + b1_ref[...], 0.0)
    p = jnp.dot(h, w_ref[...], preferred_element_type=jnp.float32)
    g2_ref[...] = p * dinv


def _t3_body(s0_ref, s1_ref, g2_ref, d0_ref, d1_ref, bc_ref, eps_ref, z_ref):
    deg = d0_ref[:, 0:1] + d1_ref[:, 0:1] + 1.0
    dinv = lax.rsqrt(jnp.maximum(deg, 1e-12))
    o = dinv * (s0_ref[...] + s1_ref[...] + g2_ref[...]) + bc_ref[...]
    mu = o[:, :D_Z]
    lv = o[:, D_Z:]
    z = mu + jnp.exp(0.5 * lv) * eps_ref[...]
    z_ref[...] = jnp.concatenate([z, jnp.zeros_like(z)], axis=1)


def _t4_body(q_ref, o_ref):
    o_ref[...] = jax.nn.sigmoid(jnp.sum(q_ref[...], axis=1, keepdims=True))


_RB = 1000
_GRID = N // _RB


def _row_spec(width):
    return pl.BlockSpec((_RB, width), lambda i: (i, 0))


def _full_spec(shape):
    return pl.BlockSpec(shape, lambda i: tuple(0 for _ in shape))


def kernel(x, edge_index, W1, b1, W_mu, b_mu, W_lv, b_lv):
    src = edge_index[0]
    dst = edge_index[1]
    pad = E_P - E
    src_a = jnp.concatenate([src, jnp.zeros((pad,), src.dtype)])
    dst_a = jnp.concatenate([dst, jnp.full((pad,), NPAD - 1, dst.dtype)])
    dst_0 = jnp.concatenate([dst, jnp.zeros((pad,), dst.dtype)])
    dst2 = dst_a.reshape(NCHP, CH)
    Wcat = jnp.concatenate([W_mu, W_lv], axis=1)
    bcat = jnp.concatenate([b_mu, b_lv], axis=0).reshape(1, 2 * D_Z)
    b1r = b1.reshape(1, D_H)
    eps = jax.random.normal(jax.random.key(42), (N, D_Z), jnp.float32)

    deg_parts = _deg_sc(dst2)
    d0 = deg_parts[0, :N]
    d1 = deg_parts[1, :N]

    g1 = pl.pallas_call(
        _t1_body,
        grid=(_GRID,),
        in_specs=[_row_spec(D_IN), _full_spec((D_IN, D_H)), _row_spec(L),
                  _row_spec(L)],
        out_specs=_row_spec(D_H),
        out_shape=jax.ShapeDtypeStruct((N, D_H), jnp.float32),
    )(x, W1, d0, d1)

    s1p = _agg_sc(g1, src_a, dst_a)
    s1 = (s1p[0, :N], s1p[1, :N])

    g2 = pl.pallas_call(
        _t2_body,
        grid=(_GRID,),
        in_specs=[_row_spec(D_H), _row_spec(D_H), _row_spec(D_H),
                  _row_spec(L), _row_spec(L), _full_spec((1, D_H)),
                  _full_spec((D_H, D_H))],
        out_specs=_row_spec(D_H),
        out_shape=jax.ShapeDtypeStruct((N, D_H), jnp.float32),
    )(s1[0], s1[1], g1, d0, d1, b1r, Wcat)

    s2p = _agg_sc(g2, src_a, dst_a)
    s2 = (s2p[0, :N], s2p[1, :N])

    z = pl.pallas_call(
        _t3_body,
        grid=(_GRID,),
        in_specs=[_row_spec(D_H), _row_spec(D_H), _row_spec(D_H),
                  _row_spec(L), _row_spec(L), _full_spec((1, D_H)),
                  _row_spec(D_Z)],
        out_specs=_row_spec(D_H),
        out_shape=jax.ShapeDtypeStruct((N, D_H), jnp.float32),
    )(s2[0], s2[1], g2, d0, d1, bcat, eps)

    qflat = _dec_sc(z, src_a, dst_0)
    q = qflat.reshape(E_P, L)

    _EB = 4096
    out = pl.pallas_call(
        _t4_body,
        grid=(E_P // _EB,),
        in_specs=[pl.BlockSpec((_EB, L), lambda i: (i, 0))],
        out_specs=pl.BlockSpec((_EB, 1), lambda i: (i, 0)),
        out_shape=jax.ShapeDtypeStruct((E_P, 1), jnp.float32),
    )(q)
    return out[:E].reshape(E)


# speedup vs baseline: 1.8366x; 1.8366x over previous
import functools

import jax
import jax.numpy as jnp
from jax import lax
from jax.experimental import pallas as pl
from jax.experimental.pallas import tpu as pltpu
from jax.experimental.pallas import tpu_sc as plsc

N = 10000
E = 320000
D_IN = 128
D_H = 128
D_Z = 64

NC = 2
NS = 16
NW = NC * NS
L = 16

CH = 128
NCHUNK = E // CH
CHUNKS_LO = NCHUNK // NW
CHUNKS_REM = NCHUNK % NW
E_P = E
NPAD = 10240
ROWS_PER_TILE = NPAD // NS

_MESH = plsc.VectorSubcoreMesh(core_axis_name="c", subcore_axis_name="s",
                               num_cores=2, num_subcores=16)


def _wid():
    return lax.axis_index("c") * NS + lax.axis_index("s")


def _nch(wid):
    return jnp.where(wid < CHUNKS_REM, CHUNKS_LO + 1, CHUNKS_LO)


@functools.partial(
    pl.kernel,
    out_type=jax.ShapeDtypeStruct((NC, NPAD, L), jnp.float32),
    mesh=_MESH,
    scratch_types=[
        pltpu.VMEM((CH,), jnp.int32),
        pltpu.VMEM((CH, L), jnp.float32),
        pltpu.VMEM((CH, L), jnp.float32),
        pltpu.VMEM_SHARED((NPAD, L), jnp.float32),
    ],
)
def _deg_sc(dst_hbm, deg_hbm, idx_v, ones_v, zb_v, acc):
    cid = lax.axis_index("c")
    sid = lax.axis_index("s")
    wid = _wid()

    def fill(r, _):
        ones_v[r, :] = jnp.full((L,), 1.0, jnp.float32)
        zb_v[r, :] = jnp.zeros((L,), jnp.float32)
        return 0

    lax.fori_loop(0, CH, fill, 0)
    for k in range(5):
        pltpu.sync_copy(
            zb_v, acc.at[pl.ds(sid * ROWS_PER_TILE + k * CH, CH)])
    plsc.subcore_barrier()

    def body(c, _):
        base = (c * NW + wid) * CH
        pltpu.sync_copy(dst_hbm.at[pl.ds(base, CH)], idx_v)
        pltpu.sync_copy(ones_v, acc.at[idx_v], add=True)
        return 0

    lax.fori_loop(0, _nch(wid), body, 0)
    plsc.subcore_barrier()
    pltpu.sync_copy(
        acc.at[pl.ds(sid * ROWS_PER_TILE, ROWS_PER_TILE)],
        deg_hbm.at[cid, pl.ds(sid * ROWS_PER_TILE, ROWS_PER_TILE)],
    )


@functools.partial(
    pl.kernel,
    out_type=jax.ShapeDtypeStruct((NC, NPAD, D_H), jnp.float32),
    mesh=_MESH,
    scratch_types=[
        pltpu.VMEM((CH,), jnp.int32),
        pltpu.VMEM((CH,), jnp.int32),
        pltpu.VMEM((CH, D_H), jnp.float32),
        pltpu.VMEM((CH, D_H), jnp.float32),
        pltpu.VMEM_SHARED((NPAD, D_H), jnp.float32),
        pltpu.SemaphoreType.DMA,
    ],
)
def _agg_sc(g_hbm, src_hbm, dst_hbm, out_hbm, idx_s, idx_d, rows_v, zb_v,
            acc, sem):
    cid = lax.axis_index("c")
    sid = lax.axis_index("s")
    wid = _wid()

    def fill(r, _):
        for c8 in range(D_H // L):
            zb_v[r, pl.ds(c8 * L, L)] = jnp.zeros((L,), jnp.float32)
        return 0

    lax.fori_loop(0, CH, fill, 0)
    for k in range(5):
        pltpu.sync_copy(
            zb_v, acc.at[pl.ds(sid * ROWS_PER_TILE + k * CH, CH)])
    plsc.subcore_barrier()

    def body(c, _):
        base = (c * NW + wid) * CH
        pltpu.sync_copy(src_hbm.at[pl.ds(base, CH)], idx_s)
        pltpu.sync_copy(dst_hbm.at[pl.ds(base, CH)], idx_d)
        pltpu.async_copy(g_hbm.at[idx_s], rows_v, sem).wait()
        pltpu.sync_copy(rows_v, acc.at[idx_d], add=True)
        return 0

    lax.fori_loop(0, _nch(wid), body, 0)
    plsc.subcore_barrier()
    pltpu.sync_copy(
        acc.at[pl.ds(sid * ROWS_PER_TILE, ROWS_PER_TILE)],
        out_hbm.at[cid, pl.ds(sid * ROWS_PER_TILE, ROWS_PER_TILE)],
    )


@functools.partial(
    pl.kernel,
    out_type=jax.ShapeDtypeStruct((E_P * L,), jnp.float32),
    mesh=_MESH,
    scratch_types=[
        pltpu.VMEM((CH,), jnp.int32),
        pltpu.VMEM((CH,), jnp.int32),
        pltpu.VMEM((CH, D_H), jnp.float32),
        pltpu.VMEM((CH, D_H), jnp.float32),
        pltpu.VMEM((CH * L,), jnp.float32),
        pltpu.SemaphoreType.DMA,
    ],
)
def _dec_sc(z_hbm, src_hbm, dst_hbm, q_hbm, idx_s, idx_d, zs_v, zd_v, q_v,
            sem):
    wid = _wid()

    def body(c, _):
        base = (c * NW + wid) * CH
        pltpu.sync_copy(src_hbm.at[pl.ds(base, CH)], idx_s)
        pltpu.sync_copy(dst_hbm.at[pl.ds(base, CH)], idx_d)
        pltpu.async_copy(z_hbm.at[idx_s], zs_v, sem).wait()
        pltpu.async_copy(z_hbm.at[idx_d], zd_v, sem).wait()

        def dot_edge(e, _):
            q = zs_v[e, pl.ds(0, L)] * zd_v[e, pl.ds(0, L)]
            for t in range(1, D_Z // L):
                q = q + zs_v[e, pl.ds(t * L, L)] * zd_v[e, pl.ds(t * L, L)]
            q_v[pl.ds(e * L, L)] = q
            return 0

        lax.fori_loop(0, CH, dot_edge, 0)
        pltpu.sync_copy(q_v, q_hbm.at[pl.ds(base * L, CH * L)])
        return 0

    lax.fori_loop(0, _nch(wid), body, 0)


def _t1_body(x_ref, w_ref, d0_ref, d1_ref, g_ref):
    deg = d0_ref[:, 0:1] + d1_ref[:, 0:1] + 1.0
    dinv = lax.rsqrt(jnp.maximum(deg, 1e-12))
    h = jnp.dot(x_ref[...], w_ref[...], preferred_element_type=jnp.float32)
    g_ref[...] = h * dinv


def _t2_body(s0_ref, s1_ref, g1_ref, d0_ref, d1_ref, b1_ref, w_ref, g2_ref):
    deg = d0_ref[:, 0:1] + d1_ref[:, 0:1] + 1.0
    dinv = lax.rsqrt(jnp.maximum(deg, 1e-12))
    h = jnp.maximum(
        dinv * (s0_ref[...] + s1_ref[...] + g1_ref[...]) + b1_ref[...], 0.0)
    p = jnp.dot(h, w_ref[...], preferred_element_type=jnp.float32)
    g2_ref[...] = p * dinv


def _t3_body(s0_ref, s1_ref, g2_ref, d0_ref, d1_ref, bc_ref, eps_ref, z_ref):
    deg = d0_ref[:, 0:1] + d1_ref[:, 0:1] + 1.0
    dinv = lax.rsqrt(jnp.maximum(deg, 1e-12))
    o = dinv * (s0_ref[...] + s1_ref[...] + g2_ref[...]) + bc_ref[...]
    mu = o[:, :D_Z]
    lv = o[:, D_Z:]
    z = mu + jnp.exp(0.5 * lv) * eps_ref[...]
    z_ref[...] = jnp.concatenate([z, jnp.zeros_like(z)], axis=1)


def _t4_body(q_ref, o_ref):
    o_ref[...] = jax.nn.sigmoid(jnp.sum(q_ref[...], axis=1, keepdims=True))


_RB = 1000
_GRID = N // _RB


def _row_spec(width):
    return pl.BlockSpec((_RB, width), lambda i: (i, 0))


def _full_spec(shape):
    return pl.BlockSpec(shape, lambda i: tuple(0 for _ in shape))


def kernel(x, edge_index, W1, b1, W_mu, b_mu, W_lv, b_lv):
    src = edge_index[0]
    dst = edge_index[1]
    Wcat = jnp.concatenate([W_mu, W_lv], axis=1)
    bcat = jnp.concatenate([b_mu, b_lv], axis=0).reshape(1, 2 * D_Z)
    b1r = b1.reshape(1, D_H)
    eps = jax.random.normal(jax.random.key(42), (N, D_Z), jnp.float32)

    deg_parts = _deg_sc(dst)
    d0 = deg_parts[0, :N]
    d1 = deg_parts[1, :N]

    g1 = pl.pallas_call(
        _t1_body,
        grid=(_GRID,),
        in_specs=[_row_spec(D_IN), _full_spec((D_IN, D_H)), _row_spec(L),
                  _row_spec(L)],
        out_specs=_row_spec(D_H),
        out_shape=jax.ShapeDtypeStruct((N, D_H), jnp.float32),
    )(x, W1, d0, d1)

    s1p = _agg_sc(g1, src, dst)
    s1 = (s1p[0, :N], s1p[1, :N])

    g2 = pl.pallas_call(
        _t2_body,
        grid=(_GRID,),
        in_specs=[_row_spec(D_H), _row_spec(D_H), _row_spec(D_H),
                  _row_spec(L), _row_spec(L), _full_spec((1, D_H)),
                  _full_spec((D_H, D_H))],
        out_specs=_row_spec(D_H),
        out_shape=jax.ShapeDtypeStruct((N, D_H), jnp.float32),
    )(s1[0], s1[1], g1, d0, d1, b1r, Wcat)

    s2p = _agg_sc(g2, src, dst)
    s2 = (s2p[0, :N], s2p[1, :N])

    z = pl.pallas_call(
        _t3_body,
        grid=(_GRID,),
        in_specs=[_row_spec(D_H), _row_spec(D_H), _row_spec(D_H),
                  _row_spec(L), _row_spec(L), _full_spec((1, D_H)),
                  _row_spec(D_Z)],
        out_specs=_row_spec(D_H),
        out_shape=jax.ShapeDtypeStruct((N, D_H), jnp.float32),
    )(s2[0], s2[1], g2, d0, d1, bcat, eps)

    qflat = _dec_sc(z, src, dst)
    q = qflat.reshape(E_P, L)

    _EB = 4000
    out = pl.pallas_call(
        _t4_body,
        grid=(E_P // _EB,),
        in_specs=[pl.BlockSpec((_EB, L), lambda i: (i, 0))],
        out_specs=pl.BlockSpec((_EB, 1), lambda i: (i, 0)),
        out_shape=jax.ShapeDtypeStruct((E_P, 1), jnp.float32),
    )(q)
    return out[:E].reshape(E)


# speedup vs baseline: 2.0384x; 1.1099x over previous
import functools

import jax
import jax.numpy as jnp
from jax import lax
from jax.experimental import pallas as pl
from jax.experimental.pallas import tpu as pltpu
from jax.experimental.pallas import tpu_sc as plsc

N = 10000
E = 320000
D_IN = 128
D_H = 128
D_Z = 64

NC = 2
NS = 16
NW = NC * NS
L = 16

CH = 128
E_P = 327680
NCHP = E_P // CH
NCH_T = NCHP // NW
NGRP = NCH_T // 8
NPAD = 10240
ROWS_PER_TILE = NPAD // NS

_MESH = plsc.VectorSubcoreMesh(core_axis_name="c", subcore_axis_name="s",
                               num_cores=2, num_subcores=16)


def _wid():
    return lax.axis_index("c") * NS + lax.axis_index("s")


@functools.partial(
    pl.kernel,
    out_type=jax.ShapeDtypeStruct((NC, NPAD, L), jnp.float32),
    mesh=_MESH,
    scratch_types=[
        pltpu.VMEM((NCH_T, CH), jnp.int32),
        pltpu.VMEM((CH, L), jnp.float32),
        pltpu.VMEM((CH, L), jnp.float32),
        pltpu.VMEM_SHARED((NPAD, L), jnp.float32),
        pltpu.SemaphoreType.DMA,
    ],
)
def _deg_sc(dst_hbm, deg_hbm, idx_all, ones_v, zb_v, acc, sem):
    cid = lax.axis_index("c")
    sid = lax.axis_index("s")
    wid = _wid()

    def fill(r, _):
        ones_v[r, :] = jnp.full((L,), 1.0, jnp.float32)
        zb_v[r, :] = jnp.zeros((L,), jnp.float32)
        return 0

    lax.fori_loop(0, CH, fill, 0)
    for k in range(NGRP):
        pltpu.sync_copy(dst_hbm.at[pl.ds((k * NW + wid) * 8, 8)],
                        idx_all.at[pl.ds(k * 8, 8)])
    for k in range(5):
        pltpu.sync_copy(
            zb_v, acc.at[pl.ds(sid * ROWS_PER_TILE + k * CH, CH)])
    plsc.subcore_barrier()

    def group(k, _):
        descs = []
        for j in range(8):
            descs.append(
                pltpu.async_copy(ones_v, acc.at[idx_all.at[k * 8 + j]], sem,
                                 add=True))
        for d in descs:
            d.wait()
        return 0

    lax.fori_loop(0, NGRP, group, 0)
    plsc.subcore_barrier()
    pltpu.sync_copy(
        acc.at[pl.ds(sid * ROWS_PER_TILE, ROWS_PER_TILE)],
        deg_hbm.at[cid, pl.ds(sid * ROWS_PER_TILE, ROWS_PER_TILE)],
    )


@functools.partial(
    pl.kernel,
    out_type=jax.ShapeDtypeStruct((NC, NPAD, D_H), jnp.float32),
    mesh=_MESH,
    scratch_types=[
        pltpu.VMEM((CH,), jnp.int32),
        pltpu.VMEM((CH,), jnp.int32),
        pltpu.VMEM((CH,), jnp.int32),
        pltpu.VMEM((CH,), jnp.int32),
        pltpu.VMEM((CH, D_H), jnp.float32),
        pltpu.VMEM((CH, D_H), jnp.float32),
        pltpu.VMEM_SHARED((NPAD, D_H), jnp.float32),
        pltpu.SemaphoreType.DMA,
        pltpu.SemaphoreType.DMA,
    ],
)
def _agg_sc(g_hbm, src_hbm, dst_hbm, out_hbm, is0, id0, is1, id1, rows0,
            rows1, acc, sm0, sm1):
    cid = lax.axis_index("c")
    sid = lax.axis_index("s")
    wid = _wid()

    def fill(r, _):
        for c8 in range(D_H // L):
            rows0[r, pl.ds(c8 * L, L)] = jnp.zeros((L,), jnp.float32)
        return 0

    lax.fori_loop(0, CH, fill, 0)
    for k in range(5):
        pltpu.sync_copy(
            rows0, acc.at[pl.ds(sid * ROWS_PER_TILE + k * CH, CH)])
    plsc.subcore_barrier()

    def pair(p, _):
        base0 = ((2 * p) * NW + wid) * CH
        base1 = ((2 * p + 1) * NW + wid) * CH
        pltpu.sync_copy(src_hbm.at[pl.ds(base0, CH)], is0)
        pltpu.sync_copy(dst_hbm.at[pl.ds(base0, CH)], id0)
        pltpu.sync_copy(src_hbm.at[pl.ds(base1, CH)], is1)
        pltpu.sync_copy(dst_hbm.at[pl.ds(base1, CH)], id1)
        d0 = pltpu.async_copy(g_hbm.at[is0], rows0, sm0)
        d1 = pltpu.async_copy(g_hbm.at[is1], rows1, sm1)
        d0.wait()
        pltpu.sync_copy(rows0, acc.at[id0], add=True)
        d1.wait()
        pltpu.sync_copy(rows1, acc.at[id1], add=True)
        return 0

    lax.fori_loop(0, NCH_T // 2, pair, 0)
    plsc.subcore_barrier()
    pltpu.sync_copy(
        acc.at[pl.ds(sid * ROWS_PER_TILE, ROWS_PER_TILE)],
        out_hbm.at[cid, pl.ds(sid * ROWS_PER_TILE, ROWS_PER_TILE)],
    )


@functools.partial(
    pl.kernel,
    out_type=jax.ShapeDtypeStruct((E_P * L,), jnp.float32),
    mesh=_MESH,
    scratch_types=[
        pltpu.VMEM((CH,), jnp.int32),
        pltpu.VMEM((CH,), jnp.int32),
        pltpu.VMEM((CH,), jnp.int32),
        pltpu.VMEM((CH,), jnp.int32),
        pltpu.VMEM((CH, D_H), jnp.float32),
        pltpu.VMEM((CH, D_H), jnp.float32),
        pltpu.VMEM((CH, D_H), jnp.float32),
        pltpu.VMEM((CH, D_H), jnp.float32),
        pltpu.VMEM((CH * L,), jnp.float32),
        pltpu.SemaphoreType.DMA,
        pltpu.SemaphoreType.DMA,
        pltpu.SemaphoreType.DMA,
        pltpu.SemaphoreType.DMA,
    ],
)
def _dec_sc(z_hbm, src_hbm, dst_hbm, q_hbm, is0, id0, is1, id1, zs0, zd0,
            zs1, zd1, q_v, sm0, sm1, sm2, sm3):
    wid = _wid()

    def dot_chunk(zs_v, zd_v):
        def dot_edge(e, _):
            q = zs_v[e, pl.ds(0, L)] * zd_v[e, pl.ds(0, L)]
            for t in range(1, D_Z // L):
                q = q + zs_v[e, pl.ds(t * L, L)] * zd_v[e, pl.ds(t * L, L)]
            q_v[pl.ds(e * L, L)] = q
            return 0

        lax.fori_loop(0, CH, dot_edge, 0)

    def pair(p, _):
        base0 = ((2 * p) * NW + wid) * CH
        base1 = ((2 * p + 1) * NW + wid) * CH
        pltpu.sync_copy(src_hbm.at[pl.ds(base0, CH)], is0)
        pltpu.sync_copy(dst_hbm.at[pl.ds(base0, CH)], id0)
        pltpu.sync_copy(src_hbm.at[pl.ds(base1, CH)], is1)
        pltpu.sync_copy(dst_hbm.at[pl.ds(base1, CH)], id1)
        d0a = pltpu.async_copy(z_hbm.at[is0], zs0, sm0)
        d0b = pltpu.async_copy(z_hbm.at[id0], zd0, sm1)
        d1a = pltpu.async_copy(z_hbm.at[is1], zs1, sm2)
        d1b = pltpu.async_copy(z_hbm.at[id1], zd1, sm3)
        d0a.wait()
        d0b.wait()
        dot_chunk(zs0, zd0)
        pltpu.sync_copy(q_v, q_hbm.at[pl.ds(base0 * L, CH * L)])
        d1a.wait()
        d1b.wait()
        dot_chunk(zs1, zd1)
        pltpu.sync_copy(q_v, q_hbm.at[pl.ds(base1 * L, CH * L)])
        return 0

    lax.fori_loop(0, NCH_T // 2, pair, 0)


def _t1_body(x_ref, w_ref, d0_ref, d1_ref, g_ref):
    deg = d0_ref[:, 0:1] + d1_ref[:, 0:1] + 1.0
    dinv = lax.rsqrt(jnp.maximum(deg, 1e-12))
    h = jnp.dot(x_ref[...], w_ref[...], preferred_element_type=jnp.float32)
    g_ref[...] = h * dinv


def _t2_body(s0_ref, s1_ref, g1_ref, d0_ref, d1_ref, b1_ref, w_ref, g2_ref):
    deg = d0_ref[:, 0:1] + d1_ref[:, 0:1] + 1.0
    dinv = lax.rsqrt(jnp.maximum(deg, 1e-12))
    h = jnp.maximum(
        dinv * (s0_ref[...] + s1_ref[...] + g1_ref[...]) + b1_ref[...], 0.0)
    p = jnp.dot(h, w_ref[...], preferred_element_type=jnp.float32)
    g2_ref[...] = p * dinv


def _t3_body(s0_ref, s1_ref, g2_ref, d0_ref, d1_ref, bc_ref, eps_ref, z_ref):
    deg = d0_ref[:, 0:1] + d1_ref[:, 0:1] + 1.0
    dinv = lax.rsqrt(jnp.maximum(deg, 1e-12))
    o = dinv * (s0_ref[...] + s1_ref[...] + g2_ref[...]) + bc_ref[...]
    mu = o[:, :D_Z]
    lv = o[:, D_Z:]
    z = mu + jnp.exp(0.5 * lv) * eps_ref[...]
    z_ref[...] = jnp.concatenate([z, jnp.zeros_like(z)], axis=1)


def _t4_body(q_ref, o_ref):
    o_ref[...] = jax.nn.sigmoid(jnp.sum(q_ref[...], axis=1, keepdims=True))


_RB = 1000
_GRID = N // _RB


def _row_spec(width):
    return pl.BlockSpec((_RB, width), lambda i: (i, 0))


def _full_spec(shape):
    return pl.BlockSpec(shape, lambda i: tuple(0 for _ in shape))


def kernel(x, edge_index, W1, b1, W_mu, b_mu, W_lv, b_lv):
    src = edge_index[0]
    dst = edge_index[1]
    pad = E_P - E
    pad_src = (jnp.arange(pad, dtype=src.dtype) * 37) % N
    pad_dst = N + (jnp.arange(pad, dtype=dst.dtype) % (NPAD - N))
    src_a = jnp.concatenate([src, pad_src])
    dst_a = jnp.concatenate([dst, pad_dst])
    dst_0 = jnp.concatenate([dst, pad_src])
    dst2 = dst_a.reshape(NCHP, CH)
    Wcat = jnp.concatenate([W_mu, W_lv], axis=1)
    bcat = jnp.concatenate([b_mu, b_lv], axis=0).reshape(1, 2 * D_Z)
    b1r = b1.reshape(1, D_H)
    eps = jax.random.normal(jax.random.key(42), (N, D_Z), jnp.float32)

    deg_parts = _deg_sc(dst2)
    d0 = deg_parts[0, :N]
    d1 = deg_parts[1, :N]

    g1 = pl.pallas_call(
        _t1_body,
        grid=(_GRID,),
        in_specs=[_row_spec(D_IN), _full_spec((D_IN, D_H)), _row_spec(L),
                  _row_spec(L)],
        out_specs=_row_spec(D_H),
        out_shape=jax.ShapeDtypeStruct((N, D_H), jnp.float32),
    )(x, W1, d0, d1)

    s1p = _agg_sc(g1, src_a, dst_a)
    s1 = (s1p[0, :N], s1p[1, :N])

    g2 = pl.pallas_call(
        _t2_body,
        grid=(_GRID,),
        in_specs=[_row_spec(D_H), _row_spec(D_H), _row_spec(D_H),
                  _row_spec(L), _row_spec(L), _full_spec((1, D_H)),
                  _full_spec((D_H, D_H))],
        out_specs=_row_spec(D_H),
        out_shape=jax.ShapeDtypeStruct((N, D_H), jnp.float32),
    )(s1[0], s1[1], g1, d0, d1, b1r, Wcat)

    s2p = _agg_sc(g2, src_a, dst_a)
    s2 = (s2p[0, :N], s2p[1, :N])

    z = pl.pallas_call(
        _t3_body,
        grid=(_GRID,),
        in_specs=[_row_spec(D_H), _row_spec(D_H), _row_spec(D_H),
                  _row_spec(L), _row_spec(L), _full_spec((1, D_H)),
                  _row_spec(D_Z)],
        out_specs=_row_spec(D_H),
        out_shape=jax.ShapeDtypeStruct((N, D_H), jnp.float32),
    )(s2[0], s2[1], g2, d0, d1, bcat, eps)

    qflat = _dec_sc(z, src_a, dst_0)
    q = qflat.reshape(E_P, L)

    _EB = 4096
    out = pl.pallas_call(
        _t4_body,
        grid=(E_P // _EB,),
        in_specs=[pl.BlockSpec((_EB, L), lambda i: (i, 0))],
        out_specs=pl.BlockSpec((_EB, 1), lambda i: (i, 0)),
        out_shape=jax.ShapeDtypeStruct((E_P, 1), jnp.float32),
    )(q)
    return out[:E].reshape(E)


# speedup vs baseline: 2.7550x; 1.3515x over previous
import functools

import jax
import jax.numpy as jnp
from jax import lax
from jax.experimental import pallas as pl
from jax.experimental.pallas import tpu as pltpu
from jax.experimental.pallas import tpu_sc as plsc

N = 10000
E = 320000
D_IN = 128
D_H = 128
D_Z = 64

NC = 2
NS = 16
NW = NC * NS
L = 16

CH = 128
E_P = 327680
NCHP = E_P // CH
NCH_T = NCHP // NW
NGRP = NCH_T // 8
NPAD = 10240
ROWS_PER_TILE = NPAD // NS

_MESH = plsc.VectorSubcoreMesh(core_axis_name="c", subcore_axis_name="s",
                               num_cores=2, num_subcores=16)


def _wid():
    return lax.axis_index("c") * NS + lax.axis_index("s")


@functools.partial(
    pl.kernel,
    out_type=jax.ShapeDtypeStruct((NC, NPAD, L), jnp.float32),
    mesh=_MESH,
    scratch_types=[
        pltpu.VMEM((NCH_T, CH), jnp.int32),
        pltpu.VMEM((CH, L), jnp.float32),
        pltpu.VMEM((CH, L), jnp.float32),
        pltpu.VMEM_SHARED((NPAD, L), jnp.float32),
        pltpu.SemaphoreType.DMA,
    ],
)
def _deg_sc(dst_hbm, deg_hbm, idx_all, ones_v, zb_v, acc, sem):
    cid = lax.axis_index("c")
    sid = lax.axis_index("s")
    wid = _wid()

    def fill(r, _):
        ones_v[r, :] = jnp.full((L,), 1.0, jnp.float32)
        zb_v[r, :] = jnp.zeros((L,), jnp.float32)
        return 0

    lax.fori_loop(0, CH, fill, 0)
    for k in range(NGRP):
        pltpu.sync_copy(dst_hbm.at[pl.ds((k * NW + wid) * 8, 8)],
                        idx_all.at[pl.ds(k * 8, 8)])
    for k in range(5):
        pltpu.sync_copy(
            zb_v, acc.at[pl.ds(sid * ROWS_PER_TILE + k * CH, CH)])
    plsc.subcore_barrier()

    def group(k, _):
        descs = []
        for j in range(8):
            descs.append(
                pltpu.async_copy(ones_v, acc.at[idx_all.at[k * 8 + j]], sem,
                                 add=True))
        for d in descs:
            d.wait()
        return 0

    lax.fori_loop(0, NGRP, group, 0)
    plsc.subcore_barrier()
    pltpu.sync_copy(
        acc.at[pl.ds(sid * ROWS_PER_TILE, ROWS_PER_TILE)],
        deg_hbm.at[cid, pl.ds(sid * ROWS_PER_TILE, ROWS_PER_TILE)],
    )


@functools.partial(
    pl.kernel,
    out_type=jax.ShapeDtypeStruct((NC, NPAD, D_H), jnp.float32),
    mesh=_MESH,
    scratch_types=[
        pltpu.VMEM((NCH_T, CH), jnp.int32),
        pltpu.VMEM((CH,), jnp.int32),
        pltpu.VMEM((CH,), jnp.int32),
        pltpu.VMEM((CH, D_H), jnp.float32),
        pltpu.VMEM((CH, D_H), jnp.float32),
        pltpu.VMEM_SHARED((NPAD, D_H), jnp.float32),
        pltpu.SemaphoreType.DMA,
        pltpu.SemaphoreType.DMA,
    ],
)
def _agg_sc(g_hbm, src_hbm, dst_hbm, out_hbm, isa, id0, id1, rows0, rows1,
            acc, sm0, sm1):
    cid = lax.axis_index("c")
    sid = lax.axis_index("s")
    wid = _wid()

    def fill(r, _):
        for c8 in range(D_H // L):
            rows0[r, pl.ds(c8 * L, L)] = jnp.zeros((L,), jnp.float32)
        return 0

    lax.fori_loop(0, CH, fill, 0)
    for k in range(NGRP):
        pltpu.sync_copy(src_hbm.at[pl.ds((k * NW + wid) * 8, 8)],
                        isa.at[pl.ds(k * 8, 8)])
    for k in range(5):
        pltpu.sync_copy(
            rows0, acc.at[pl.ds(sid * ROWS_PER_TILE + k * CH, CH)])
    plsc.subcore_barrier()

    def pair(p, _):
        r = 2 * p
        g0 = ((r // 8) * NW + wid) * 8 + (r % 8)
        g1 = g0 + 1
        d0 = pltpu.async_copy(g_hbm.at[isa.at[r]], rows0, sm0)
        d1 = pltpu.async_copy(g_hbm.at[isa.at[r + 1]], rows1, sm1)
        pltpu.sync_copy(dst_hbm.at[g0], id0)
        pltpu.sync_copy(dst_hbm.at[g1], id1)
        d0.wait()
        pltpu.sync_copy(rows0, acc.at[id0], add=True)
        d1.wait()
        pltpu.sync_copy(rows1, acc.at[id1], add=True)
        return 0

    lax.fori_loop(0, NCH_T // 2, pair, 0)
    plsc.subcore_barrier()
    pltpu.sync_copy(
        acc.at[pl.ds(sid * ROWS_PER_TILE, ROWS_PER_TILE)],
        out_hbm.at[cid, pl.ds(sid * ROWS_PER_TILE, ROWS_PER_TILE)],
    )


@functools.partial(
    pl.kernel,
    out_type=jax.ShapeDtypeStruct((E_P * L,), jnp.float32),
    mesh=_MESH,
    scratch_types=[
        pltpu.VMEM((NCH_T, CH), jnp.int32),
        pltpu.VMEM((NCH_T, CH), jnp.int32),
        pltpu.VMEM((CH, D_H), jnp.float32),
        pltpu.VMEM((CH, D_H), jnp.float32),
        pltpu.VMEM((CH, D_H), jnp.float32),
        pltpu.VMEM((CH, D_H), jnp.float32),
        pltpu.VMEM((CH * L,), jnp.float32),
        pltpu.SemaphoreType.DMA,
        pltpu.SemaphoreType.DMA,
        pltpu.SemaphoreType.DMA,
        pltpu.SemaphoreType.DMA,
    ],
)
def _dec_sc(z_hbm, src_hbm, dst_hbm, q_hbm, isa, ida, zs0, zd0, zs1, zd1,
            q_v, sm0, sm1, sm2, sm3):
    wid = _wid()

    for k in range(NGRP):
        pltpu.sync_copy(src_hbm.at[pl.ds((k * NW + wid) * 8, 8)],
                        isa.at[pl.ds(k * 8, 8)])
        pltpu.sync_copy(dst_hbm.at[pl.ds((k * NW + wid) * 8, 8)],
                        ida.at[pl.ds(k * 8, 8)])

    def dot_chunk(zs_v, zd_v):
        @functools.partial(plsc.parallel_loop, 0, CH, unroll=4)
        def dot_edge(e):
            q = zs_v[e, pl.ds(0, L)] * zd_v[e, pl.ds(0, L)]
            for t in range(1, D_Z // L):
                q = q + zs_v[e, pl.ds(t * L, L)] * zd_v[e, pl.ds(t * L, L)]
            q_v[pl.ds(e * L, L)] = q

    def pair(p, _):
        r = 2 * p
        g0 = ((r // 8) * NW + wid) * 8 + (r % 8)
        g1 = g0 + 1
        d0a = pltpu.async_copy(z_hbm.at[isa.at[r]], zs0, sm0)
        d0b = pltpu.async_copy(z_hbm.at[ida.at[r]], zd0, sm1)
        d1a = pltpu.async_copy(z_hbm.at[isa.at[r + 1]], zs1, sm2)
        d1b = pltpu.async_copy(z_hbm.at[ida.at[r + 1]], zd1, sm3)
        d0a.wait()
        d0b.wait()
        dot_chunk(zs0, zd0)
        pltpu.sync_copy(q_v, q_hbm.at[pl.ds(g0 * CH * L, CH * L)])
        d1a.wait()
        d1b.wait()
        dot_chunk(zs1, zd1)
        pltpu.sync_copy(q_v, q_hbm.at[pl.ds(g1 * CH * L, CH * L)])
        return 0

    lax.fori_loop(0, NCH_T // 2, pair, 0)


def _t1_body(x_ref, w_ref, d0_ref, d1_ref, g_ref):
    deg = d0_ref[:, 0:1] + d1_ref[:, 0:1] + 1.0
    dinv = lax.rsqrt(jnp.maximum(deg, 1e-12))
    h = jnp.dot(x_ref[...], w_ref[...], preferred_element_type=jnp.float32)
    g_ref[...] = h * dinv


def _t2_body(s0_ref, s1_ref, g1_ref, d0_ref, d1_ref, b1_ref, w_ref, g2_ref):
    deg = d0_ref[:, 0:1] + d1_ref[:, 0:1] + 1.0
    dinv = lax.rsqrt(jnp.maximum(deg, 1e-12))
    h = jnp.maximum(
        dinv * (s0_ref[...] + s1_ref[...] + g1_ref[...]) + b1_ref[...], 0.0)
    p = jnp.dot(h, w_ref[...], preferred_element_type=jnp.float32)
    g2_ref[...] = p * dinv


def _t3_body(s0_ref, s1_ref, g2_ref, d0_ref, d1_ref, bc_ref, eps_ref, z_ref):
    deg = d0_ref[:, 0:1] + d1_ref[:, 0:1] + 1.0
    dinv = lax.rsqrt(jnp.maximum(deg, 1e-12))
    o = dinv * (s0_ref[...] + s1_ref[...] + g2_ref[...]) + bc_ref[...]
    mu = o[:, :D_Z]
    lv = o[:, D_Z:]
    z = mu + jnp.exp(0.5 * lv) * eps_ref[...]
    z_ref[...] = jnp.concatenate([z, jnp.zeros_like(z)], axis=1)


def _t4_body(q_ref, o_ref):
    o_ref[...] = jax.nn.sigmoid(jnp.sum(q_ref[...], axis=1, keepdims=True))


_RB = 1000
_GRID = N // _RB


def _row_spec(width):
    return pl.BlockSpec((_RB, width), lambda i: (i, 0))


def _full_spec(shape):
    return pl.BlockSpec(shape, lambda i: tuple(0 for _ in shape))


def kernel(x, edge_index, W1, b1, W_mu, b_mu, W_lv, b_lv):
    src = edge_index[0]
    dst = edge_index[1]
    pad = E_P - E
    pad_src = (jnp.arange(pad, dtype=src.dtype) * 37) % N
    pad_dst = N + (jnp.arange(pad, dtype=dst.dtype) % (NPAD - N))
    src_a = jnp.concatenate([src, pad_src])
    dst_a = jnp.concatenate([dst, pad_dst])
    dst_0 = jnp.concatenate([dst, pad_src])
    src2 = src_a.reshape(NCHP, CH)
    dst2 = dst_a.reshape(NCHP, CH)
    dst2_0 = dst_0.reshape(NCHP, CH)
    Wcat = jnp.concatenate([W_mu, W_lv], axis=1)
    bcat = jnp.concatenate([b_mu, b_lv], axis=0).reshape(1, 2 * D_Z)
    b1r = b1.reshape(1, D_H)
    eps = jax.random.normal(jax.random.key(42), (N, D_Z), jnp.float32)

    deg_parts = _deg_sc(dst2)
    d0 = deg_parts[0, :N]
    d1 = deg_parts[1, :N]

    g1 = pl.pallas_call(
        _t1_body,
        grid=(_GRID,),
        in_specs=[_row_spec(D_IN), _full_spec((D_IN, D_H)), _row_spec(L),
                  _row_spec(L)],
        out_specs=_row_spec(D_H),
        out_shape=jax.ShapeDtypeStruct((N, D_H), jnp.float32),
    )(x, W1, d0, d1)

    s1p = _agg_sc(g1, src2, dst2)
    s1 = (s1p[0, :N], s1p[1, :N])

    g2 = pl.pallas_call(
        _t2_body,
        grid=(_GRID,),
        in_specs=[_row_spec(D_H), _row_spec(D_H), _row_spec(D_H),
                  _row_spec(L), _row_spec(L), _full_spec((1, D_H)),
                  _full_spec((D_H, D_H))],
        out_specs=_row_spec(D_H),
        out_shape=jax.ShapeDtypeStruct((N, D_H), jnp.float32),
    )(s1[0], s1[1], g1, d0, d1, b1r, Wcat)

    s2p = _agg_sc(g2, src2, dst2)
    s2 = (s2p[0, :N], s2p[1, :N])

    z = pl.pallas_call(
        _t3_body,
        grid=(_GRID,),
        in_specs=[_row_spec(D_H), _row_spec(D_H), _row_spec(D_H),
                  _row_spec(L), _row_spec(L), _full_spec((1, D_H)),
                  _row_spec(D_Z)],
        out_specs=_row_spec(D_H),
        out_shape=jax.ShapeDtypeStruct((N, D_H), jnp.float32),
    )(s2[0], s2[1], g2, d0, d1, bcat, eps)

    qflat = _dec_sc(z, src2, dst2_0)
    q = qflat.reshape(E_P, L)

    _EB = 4096
    out = pl.pallas_call(
        _t4_body,
        grid=(E_P // _EB,),
        in_specs=[pl.BlockSpec((_EB, L), lambda i: (i, 0))],
        out_specs=pl.BlockSpec((_EB, 1), lambda i: (i, 0)),
        out_shape=jax.ShapeDtypeStruct((E_P, 1), jnp.float32),
    )(q)
    return out[:E].reshape(E)
